# Initial kernel scaffold; baseline (speedup 1.0000x reference)
import jax
import jax.numpy as jnp
from jax.experimental import pallas as pl


def kernel(x, edge_index, params):
    raise NotImplementedError("write your pallas kernel here")


# speedup vs baseline: 7.9807x; 7.9807x over previous
import functools

import jax
import jax.numpy as jnp
import numpy as np
from jax.experimental import pallas as pl
from jax.experimental.pallas import tpu as pltpu

H = 128
L = 3
GH = 256
B = 8
T = 12
N = 325
E = 2600

NPAD = 352
EPAD = 2608
TPAD = 16
P = 336
NROWS = B * N
NRPAD = 2688


def _lstm_body(x_ref, w0_ref, wh_ref, wi_ref, b_ref, out_ref, xsa, xsb):
    f32 = jnp.float32

    def run_dir(l, d, xs_in, xs_out):
        wh = wh_ref[2 * l + d]
        bias = b_ref[2 * l + d][0:1]
        if l == 0:
            w0 = w0_ref[d][0:1]
        else:
            wi = wi_ref[2 * (l - 1) + d]

        h0 = jnp.zeros((P, H), f32)
        c0 = jnp.zeros((P, H), f32)
        acc0 = jnp.zeros((P, H), f32)

        def cell(g, c):
            ig = jax.nn.sigmoid(g[:, :H])
            fg = jax.nn.sigmoid(g[:, H:2 * H])
            gg = jnp.tanh(g[:, 2 * H:3 * H])
            og = jax.nn.sigmoid(g[:, 3 * H:])
            c2 = fg * c + ig * gg
            h2 = og * jnp.tanh(c2)
            return h2, c2

        if l == 0:
            h, c = h0, c0
            for s in range(T):
                t = s if d == 0 else T - 1 - s
                xcol = x_ref[:, t:t + 1]
                g = xcol * w0 + jnp.dot(h, wh, preferred_element_type=f32) + bias
                h, c = cell(g, c)
                xs_out[t, :, d * H:(d + 1) * H] = h
            return None

        def step(s, carry):
            h, c, acc = carry
            t = s if d == 0 else T - 1 - s
            xin = xs_in[t]
            gih = jnp.dot(xin, wi, preferred_element_type=f32)
            g = gih + jnp.dot(h, wh, preferred_element_type=f32) + bias
            h2, c2 = cell(g, c)
            if l < L - 1:
                xs_out[t, :, d * H:(d + 1) * H] = h2
                acc2 = acc
            else:
                acc2 = acc + h2
            return (h2, c2, acc2)

        _, _, acc = jax.lax.fori_loop(0, T, step, (h0, c0, acc0))
        return acc

    for l in range(L):
        xs_in, xs_out = (xsa, xsb) if l % 2 == 1 else (xsb, xsa)
        if l < L - 1:
            run_dir(l, 0, xs_in, xs_out)
            run_dir(l, 1, xs_in, xs_out)
        else:
            accf = run_dir(l, 0, xs_in, xs_out)
            accb = run_dir(l, 1, xs_in, xs_out)
            inv_t = f32(1.0 / T)
            out_ref[:, :H] = accf * inv_t
            out_ref[:, H:] = accb * inv_t


def _lstm_stage(xtp, w0, wh, wi, bb):
    grid = NRPAD // P
    return pl.pallas_call(
        _lstm_body,
        grid=(grid,),
        in_specs=[
            pl.BlockSpec((P, TPAD), lambda i: (i, 0)),
            pl.BlockSpec((2, 8, 4 * H), lambda i: (0, 0, 0)),
            pl.BlockSpec((2 * L, H, 4 * H), lambda i: (0, 0, 0)),
            pl.BlockSpec((2 * (L - 1), 2 * H, 4 * H), lambda i: (0, 0, 0)),
            pl.BlockSpec((2 * L, 8, 4 * H), lambda i: (0, 0, 0)),
        ],
        out_specs=pl.BlockSpec((P, 2 * H), lambda i: (i, 0)),
        out_shape=jax.ShapeDtypeStruct((NRPAD, 2 * H), jnp.float32),
        scratch_shapes=[
            pltpu.VMEM((T, P, 2 * H), jnp.float32),
            pltpu.VMEM((T, P, 2 * H), jnp.float32),
        ],
    )(xtp, w0, wh, wi, bb)


def _adj_body(edges_ref, c_ref):
    row = edges_ref[:, 0:1]
    col = edges_ref[:, 1:2]
    iota = jax.lax.broadcasted_iota(jnp.int32, (EPAD, NPAD), 1)
    m_row = (iota == row).astype(jnp.float32)
    m_col = (iota == col).astype(jnp.float32)
    c_ref[...] = jax.lax.dot_general(
        m_col, m_row, (((0,), (0,)), ((), ())),
        preferred_element_type=jnp.float32)


def _adj_stage(edges_p):
    return pl.pallas_call(
        _adj_body,
        out_shape=jax.ShapeDtypeStruct((NPAD, NPAD), jnp.float32),
    )(edges_p)


def _gcn_body(feats_ref, c_ref, gw_ref, gb_ref, nw_ref, nb_ref, na_ref,
              cw_ref, cb_ref, out_ref):
    f32 = jnp.float32
    rmask1 = (jax.lax.broadcasted_iota(jnp.int32, (NPAD, 1), 0) < N)
    rmaskf = rmask1.astype(f32)
    ii = jax.lax.broadcasted_iota(jnp.int32, (NPAD, NPAD), 0)
    jj = jax.lax.broadcasted_iota(jnp.int32, (NPAD, NPAD), 1)

    c = jnp.where((ii < N) & (jj < N), c_ref[...], f32(0.0))
    deg = jnp.sum(c, axis=1, keepdims=True) + 1.0
    dis = jax.lax.rsqrt(deg)
    eye = jnp.where((ii == jj) & (ii < N), f32(1.0), f32(0.0))
    eye_full = jnp.where(ii == jj, f32(1.0), f32(0.0))
    a_hat = jnp.dot((c + eye) * dis, eye_full * dis,
                    preferred_element_type=f32)

    inv_n = f32(1.0 / N)
    h = feats_ref[...]
    for j in range(3):
        hin = h
        w = gw_ref[j]
        bias = gb_ref[j][0:1]
        hw = jnp.dot(h.reshape(B * NPAD, GH), w,
                     preferred_element_type=f32).reshape(B, NPAD, GH)
        agg = jnp.stack(
            [jnp.dot(a_hat, hw[bb], preferred_element_type=f32)
             for bb in range(B)], axis=0)
        x = (agg + bias) * rmaskf
        mean = jnp.sum(x, axis=1) * inv_n
        xm = (x - na_ref[j][0:1] * mean[:, None, :]) * rmaskf
        var = jnp.sum(xm * xm, axis=1) * inv_n
        xm = xm / jnp.sqrt(var + 1e-5)[:, None, :]
        gn = xm * nw_ref[j][0:1] + nb_ref[j][0:1]
        h = jnp.maximum(gn * rmaskf + hin, 0.0)

    pooled = jnp.sum(h, axis=1) * inv_n
    logits = jnp.dot(pooled, cw_ref[...],
                     preferred_element_type=f32) + cb_ref[0, 0]
    out_ref[...] = jnp.broadcast_to(logits[:, 0:1], (B, 128))


def _gcn_stage(feats, c, gw, gb, nw, nb, na, cw, cb):
    return pl.pallas_call(
        _gcn_body,
        out_shape=jax.ShapeDtypeStruct((B, 128), jnp.float32),
    )(feats, c, gw, gb, nw, nb, na, cw, cb)


@jax.jit
def kernel(x, edge_index, params):
    f32 = jnp.float32
    p = params

    w0 = jnp.zeros((2, 8, 4 * H), f32)
    w0 = w0.at[0, 0].set(p["W_ih_l0_fwd"][:, 0])
    w0 = w0.at[1, 0].set(p["W_ih_l0_bwd"][:, 0])
    wh = jnp.stack([p["W_hh_l%d_%s" % (l, d)].T
                    for l in range(L) for d in ("fwd", "bwd")], axis=0)
    wi = jnp.stack([p["W_ih_l%d_%s" % (l, d)].T
                    for l in range(1, L) for d in ("fwd", "bwd")], axis=0)
    bb = jnp.zeros((2 * L, 8, 4 * H), f32)
    for l in range(L):
        for di, d in enumerate(("fwd", "bwd")):
            bb = bb.at[2 * l + di, 0].set(
                p["b_ih_l%d_%s" % (l, d)] + p["b_hh_l%d_%s" % (l, d)])

    xtp = jnp.transpose(x, (0, 2, 1)).reshape(NROWS, T)
    xtp = jnp.pad(xtp, ((0, NRPAD - NROWS), (0, TPAD - T)))

    node_feats = _lstm_stage(xtp, w0, wh, wi, bb)[:NROWS]
    feats = jnp.pad(node_feats.reshape(B, N, 2 * H),
                    ((0, 0), (0, NPAD - N), (0, 0)))

    edges_p = jnp.full((EPAD, 8), NPAD - 1, jnp.int32)
    edges_p = edges_p.at[:E, 0:2].set(edge_index.astype(jnp.int32).T)
    c = _adj_stage(edges_p)

    gw = jnp.stack([p["gcn%d_W" % (j + 1)].T for j in range(3)], axis=0)
    gb = jnp.zeros((3, 8, GH), f32)
    nw = jnp.zeros((3, 8, GH), f32)
    nb = jnp.zeros((3, 8, GH), f32)
    na = jnp.zeros((3, 8, GH), f32)
    for j in range(3):
        gb = gb.at[j, 0].set(p["gcn%d_b" % (j + 1)])
        nw = nw.at[j, 0].set(p["norm%d_w" % (j + 1)])
        nb = nb.at[j, 0].set(p["norm%d_b" % (j + 1)])
        na = na.at[j, 0].set(p["norm%d_a" % (j + 1)])
    cw = jnp.zeros((GH, 128), f32).at[:, 0].set(p["cls_W"][0])
    cb = jnp.broadcast_to(p["cls_b"].reshape(1, 1), (8, 128))

    out = _gcn_stage(feats, c, gw, gb, nw, nb, na, cw, cb)
    return out[:, 0:1]
